# baseline (speedup 1.0000x reference)
import jax
import jax.numpy as jnp
from jax import lax
from jax.experimental import pallas as pl
from jax.experimental.pallas import tpu as pltpu
from jax.experimental.pallas import tpu_sc as plsc

N_NODES = 10000
N_EDGES = 160000
F = 16
NEG_SLOPE = 0.01

NC, NS = 2, 16
NW = NC * NS
RQ = N_EDGES // 4
RB = 1600
NBLK = RQ // RB
BLK_E = 4 * RB
WR = 1248
WR_LAST = RQ - (NW - 1) * WR
RCH = 32
NCH = WR // RCH
NCH_LAST = WR_LAST // RCH
ROWS2 = 2 * N_NODES
SROWS = ROWS2 + 96
ZPT = SROWS // NS
WPT = 1248
PAYW = 2 * F

_mesh = plsc.VectorSubcoreMesh(
    core_axis_name="c", subcore_axis_name="s", num_cores=NC, num_subcores=NS)
_sc_params = pltpu.CompilerParams(use_tc_tiling_on_sc=False)


def _gather_body(feat_h, src_h, dst_h, hz_f,
                 idxs_v, idxd_v, gis0, gis1, gid0, gid1,
                 gh0, gh1, gz0, gz1, hzf0, hzf1,
                 sh0, sh1, sz0, sz1, so0, so1):
  c = lax.axis_index("c")
  s = lax.axis_index("s")
  w = c * NS + s
  bw = w * WR
  ifirst = bw // RB

  for ib in range(2):
    ibl = jnp.minimum(ifirst + ib, NBLK - 1)
    for j in range(4):
      pltpu.sync_copy(src_h.at[pl.ds(ibl * BLK_E + j * RB, RB)],
                      idxs_v.at[pl.ds((ib * 4 + j) * RB, RB)])
      pltpu.sync_copy(dst_h.at[pl.ds(ibl * BLK_E + j * RB, RB)],
                      idxd_v.at[pl.ds((ib * 4 + j) * RB, RB)])

  gis = (gis0, gis1)
  gid = (gid0, gid1)
  gh = (gh0, gh1)
  gz = (gz0, gz1)
  hzf = (hzf0, hzf1)
  hsem = (sh0, sh1)
  zsem = (sz0, sz1)
  osem = (so0, so1)

  def build_gidx(i, b):
    g0 = bw + i * RCH
    blk = g0 // RB
    base = (blk - ifirst) * 4 * RB + (g0 - blk * RB)
    for j in range(4):
      for h2 in range(2):
        off = base + j * RB + 16 * h2
        gis[b][pl.ds(32 * j + 16 * h2, 16)] = idxs_v[pl.ds(off, 16)]
        gid[b][pl.ds(32 * j + 16 * h2, 16)] = idxd_v[pl.ds(off, 16)]

  def start_gather(b):
    pltpu.async_copy(feat_h.at[gis[b]], gh[b], hsem[b])
    pltpu.async_copy(feat_h.at[gid[b]], gz[b], zsem[b])

  def wait_gather(b):
    pltpu.make_async_copy(feat_h.at[gis[b]], gh[b], hsem[b]).wait()
    pltpu.make_async_copy(feat_h.at[gid[b]], gz[b], zsem[b]).wait()

  def pack(b):
    for i2 in range(RCH):
      for j in range(4):
        hzf[b][pl.ds(128 * i2 + 32 * j, 16)] = gh[b][32 * j + i2, pl.ds(0, 16)]
        hzf[b][pl.ds(128 * i2 + 32 * j + 16, 16)] = gz[b][32 * j + i2, pl.ds(0, 16)]

  def start_out(i, b):
    pltpu.async_copy(
        hzf[b], hz_f.at[pl.ds((bw + i * RCH) * 128, RCH * 128)], osem[b])

  def wait_out(b):
    pltpu.make_async_copy(
        hzf[b], hz_f.at[pl.ds(bw * 128, RCH * 128)], osem[b]).wait()

  build_gidx(0, 0)
  start_gather(0)
  build_gidx(1, 1)
  start_gather(1)

  def pair(k, _):
    for b in range(2):
      i = 2 * k + b
      wait_gather(b)

      @pl.when(k > 0)
      def _wo():
        wait_out(b)

      pack(b)
      start_out(i, b)
      nxt = i + 2
      if b == 0:
        build_gidx(nxt, b)
        start_gather(b)
      else:
        @pl.when(k < (NCH - 1) // 2 - 1)
        def _ng():
          build_gidx(nxt, b)
          start_gather(b)
    return 0

  lax.fori_loop(0, (NCH - 1) // 2, pair, 0)

  wait_gather(0)
  wait_out(0)
  pack(0)
  start_out(NCH - 1, 0)
  wait_out(1)
  wait_out(0)

  @pl.when(w == NW - 1)
  def _extra():
    for i in (NCH, NCH + 1):
      build_gidx(i, 0)
      start_gather(0)
      wait_gather(0)
      pack(0)
      pltpu.sync_copy(hzf[0], hz_f.at[pl.ds((bw + i * RCH) * 128, RCH * 128)])


def _sc_gather(feat, src, dst):
  return pl.kernel(
      _gather_body,
      out_type=jax.ShapeDtypeStruct((N_EDGES * 32,), jnp.float32),
      mesh=_mesh,
      compiler_params=_sc_params,
      scratch_types=[
          pltpu.VMEM((8 * RB,), jnp.int32),
          pltpu.VMEM((8 * RB,), jnp.int32),
          pltpu.VMEM((128,), jnp.int32),
          pltpu.VMEM((128,), jnp.int32),
          pltpu.VMEM((128,), jnp.int32),
          pltpu.VMEM((128,), jnp.int32),
          pltpu.VMEM((128, F), jnp.float32),
          pltpu.VMEM((128, F), jnp.float32),
          pltpu.VMEM((128, F), jnp.float32),
          pltpu.VMEM((128, F), jnp.float32),
          pltpu.VMEM((RCH * 128,), jnp.float32),
          pltpu.VMEM((RCH * 128,), jnp.float32),
      ] + [pltpu.SemaphoreType.DMA] * 6,
  )(feat, src, dst)


def _k2_body(hz_ref, ef_ref, et_ref,
             wa_ref, ba_ref, wc0_ref, wc1_ref, bc0_ref, bc1_ref,
             rm_ref, tm_ref, out_ref):
  x = hz_ref[...].reshape(RB, 128)
  a = jnp.dot(x, wa_ref[...], preferred_element_type=jnp.float32) + ba_ref[...]
  a = jnp.where(a >= 0.0, a, NEG_SLOPE * a)
  exa = jnp.exp(a)
  efa = ef_ref[...]
  eta = et_ref[...]
  lane = lax.broadcasted_iota(jnp.int32, (1, F), 1)
  onehot = (lane == 0).astype(jnp.float32)
  parts = []
  for j in range(4):
    h = x[:, 32 * j:32 * j + F]
    ef = efa[j * RB:(j + 1) * RB]
    p = (jnp.dot(ef, rm_ref[...], preferred_element_type=jnp.float32) *
         jnp.dot(h, tm_ref[...], preferred_element_type=jnp.float32))
    v0 = (jnp.dot(p, wc0_ref[...], preferred_element_type=jnp.float32) +
          jnp.dot(h, bc0_ref[...], preferred_element_type=jnp.float32))
    v1 = (jnp.dot(p, wc1_ref[...], preferred_element_type=jnp.float32) +
          jnp.dot(h, bc1_ref[...], preferred_element_type=jnp.float32))
    m0 = (eta[j * RB:(j + 1) * RB] == 0).astype(jnp.float32)
    ex = exa[:, F * j:F * (j + 1)]
    msg = ex * (m0 * v0 + (1.0 - m0) * v1)
    exl = ex * onehot
    parts.append(msg)
    parts.append(exl)
  out = jnp.concatenate(parts, axis=1)
  out_ref[...] = out.reshape(RB * 128)


def _tc_dense(hz_f, efeat, et8, wa, ba, wc0, wc1, bc0, bc1, rm, tm):
  return pl.pallas_call(
      _k2_body,
      grid=(NBLK,),
      in_specs=[
          pl.BlockSpec((RB * 128,), lambda i: (i,)),
          pl.BlockSpec((BLK_E, F), lambda i: (i, 0)),
          pl.BlockSpec((BLK_E, 1), lambda i: (i, 0)),
          pl.BlockSpec((128, 4 * F), lambda i: (0, 0)),
          pl.BlockSpec((1, 1), lambda i: (0, 0)),
          pl.BlockSpec((F * F, F), lambda i: (0, 0)),
          pl.BlockSpec((F * F, F), lambda i: (0, 0)),
          pl.BlockSpec((F, F), lambda i: (0, 0)),
          pl.BlockSpec((F, F), lambda i: (0, 0)),
          pl.BlockSpec((F, F * F), lambda i: (0, 0)),
          pl.BlockSpec((F, F * F), lambda i: (0, 0)),
      ],
      out_specs=pl.BlockSpec((RB * 128,), lambda i: (i,)),
      out_shape=jax.ShapeDtypeStruct((N_EDGES * 32,), jnp.float32),
  )(hz_f, efeat, et8, wa, ba, wc0, wc1, bc0, bc1, rm, tm)


def _scatter_body(pay_f, dst_h, et_h, spart_f,
                  idxd_v, idxt_v, r0, r1, pf0, pf1, sb0, sb1,
                  wb2d, wbf, sacc, ps0, ps1, cs0, cs1):
  c = lax.axis_index("c")
  s = lax.axis_index("s")
  w = c * NS + s
  bw = w * WR
  ifirst = bw // RB

  for i in range(128):
    sb0[i, pl.ds(0, 16)] = jnp.zeros((16,), jnp.float32)
    sb0[i, pl.ds(16, 16)] = jnp.zeros((16,), jnp.float32)
  for m in range(9):
    pltpu.sync_copy(sb0.at[pl.ds(0, 128)],
                    sacc.at[pl.ds(s * ZPT + m * 128, 128)])
  pltpu.sync_copy(sb0.at[pl.ds(0, ZPT - 9 * 128)],
                  sacc.at[pl.ds(s * ZPT + 9 * 128, ZPT - 9 * 128)])

  plsc.subcore_barrier()

  for ib in range(2):
    ibl = jnp.minimum(ifirst + ib, NBLK - 1)
    for j in range(4):
      pltpu.sync_copy(dst_h.at[pl.ds(ibl * BLK_E + j * RB, RB)],
                      idxd_v.at[pl.ds((ib * 4 + j) * RB, RB)])
      pltpu.sync_copy(et_h.at[pl.ds(ibl * BLK_E + j * RB, RB)],
                      idxt_v.at[pl.ds((ib * 4 + j) * RB, RB)])

  rb = (r0, r1)
  pf = (pf0, pf1)
  sb = (sb0, sb1)
  psem = (ps0, ps1)
  csem = (cs0, cs1)

  def build_ridx(i, b):
    g0 = bw + i * RCH
    blk = g0 // RB
    base = (blk - ifirst) * 4 * RB + (g0 - blk * RB)
    for j in range(4):
      for h2 in range(2):
        off = base + j * RB + 16 * h2
        d16 = idxd_v[pl.ds(off, 16)]
        t16 = idxt_v[pl.ds(off, 16)]
        rb[b][pl.ds(32 * j + 16 * h2, 16)] = d16 + t16 * N_NODES

  def start_pay(i, b):
    pltpu.async_copy(
        pay_f.at[pl.ds((bw + i * RCH) * 128, RCH * 128)], pf[b], psem[b])

  def wait_pay(b):
    pltpu.make_async_copy(
        pay_f.at[pl.ds(bw * 128, RCH * 128)], pf[b], psem[b]).wait()

  def unpack(b):
    for i2 in range(RCH):
      for j in range(4):
        sb[b][32 * j + i2, pl.ds(0, 16)] = pf[b][pl.ds(128 * i2 + 32 * j, 16)]
        sb[b][32 * j + i2, pl.ds(16, 16)] = pf[b][pl.ds(128 * i2 + 32 * j + 16, 16)]

  def start_scat(b):
    pltpu.async_copy(sb[b], sacc.at[rb[b]], csem[b], add=True)

  def wait_scat(b):
    pltpu.make_async_copy(sb[b], sacc.at[rb[b]], csem[b]).wait()

  start_pay(0, 0)
  start_pay(1, 1)

  def pair(k, _):
    for b in range(2):
      i = 2 * k + b
      wait_pay(b)

      @pl.when(k > 0)
      def _ws():
        wait_scat(b)

      unpack(b)
      build_ridx(i, b)
      start_scat(b)
      nxt = i + 2
      if b == 0:
        start_pay(nxt, b)
      else:
        @pl.when(k < (NCH - 1) // 2 - 1)
        def _np():
          start_pay(nxt, b)
    return 0

  lax.fori_loop(0, (NCH - 1) // 2, pair, 0)

  wait_pay(0)
  wait_scat(0)
  unpack(0)
  build_ridx(NCH - 1, 0)
  start_scat(0)
  wait_scat(1)
  wait_scat(0)

  @pl.when(w == NW - 1)
  def _extra():
    for i in (NCH, NCH + 1):
      start_pay(i, 0)
      wait_pay(0)
      unpack(0)
      build_ridx(i, 0)
      pltpu.sync_copy(sb[0], sacc.at[rb[0]], add=True)

  plsc.subcore_barrier()

  def wb_chunk(q0, nrows):
    pltpu.sync_copy(sacc.at[pl.ds(q0, nrows)], wb2d.at[pl.ds(0, nrows)])
    for r in range(nrows):
      wbf[pl.ds(32 * r, 16)] = wb2d[r, pl.ds(0, 16)]
      wbf[pl.ds(32 * r + 16, 16)] = wb2d[r, pl.ds(16, 16)]
    pltpu.sync_copy(wbf.at[pl.ds(0, nrows * 32)],
                    spart_f.at[pl.ds((c * ROWS2 + q0) * 32, nrows * 32)])

  @pl.when(s < NS - 1)
  def _wb():
    def wbody(m, _):
      wb_chunk(s * WPT + m * 96, 96)
      return 0
    lax.fori_loop(0, 13, wbody, 0)

  @pl.when(s == NS - 1)
  def _wb_last():
    def wbody(m, _):
      wb_chunk((NS - 1) * WPT + m * 128, 128)
      return 0
    lax.fori_loop(0, 10, wbody, 0)


def _sc_scatter(payload_f, dst, et):
  return pl.kernel(
      _scatter_body,
      out_type=jax.ShapeDtypeStruct((NC * ROWS2 * PAYW,), jnp.float32),
      mesh=_mesh,
      compiler_params=_sc_params,
      scratch_types=[
          pltpu.VMEM((8 * RB,), jnp.int32),
          pltpu.VMEM((8 * RB,), jnp.int32),
          pltpu.VMEM((128,), jnp.int32),
          pltpu.VMEM((128,), jnp.int32),
          pltpu.VMEM((RCH * 128,), jnp.float32),
          pltpu.VMEM((RCH * 128,), jnp.float32),
          pltpu.VMEM((128, PAYW), jnp.float32),
          pltpu.VMEM((128, PAYW), jnp.float32),
          pltpu.VMEM((128, PAYW), jnp.float32),
          pltpu.VMEM((128 * PAYW,), jnp.float32),
          pltpu.VMEM_SHARED((SROWS, PAYW), jnp.float32),
      ] + [pltpu.SemaphoreType.DMA] * 4,
  )(payload_f, dst, et)


def _k4_body(sp0_ref, sp1_ref, bias_ref, out_ref):
  x = (sp0_ref[...].reshape(ROWS2 // 4, 128) +
       sp1_ref[...].reshape(ROWS2 // 4, 128))
  x0 = x[0:N_NODES // 4]
  x1 = x[N_NODES // 4:2 * (N_NODES // 4)]
  for u in range(4):
    m0 = x0[:, 32 * u:32 * u + F]
    d0 = x0[:, 32 * u + F:32 * u + F + 1]
    m1 = x1[:, 32 * u:32 * u + F]
    d1 = x1[:, 32 * u + F:32 * u + F + 1]
    out_ref[:, F * u:F * (u + 1)] = (m0 / jnp.where(d0 > 0, d0, 1.0) +
                                     m1 / jnp.where(d1 > 0, d1, 1.0) +
                                     bias_ref[...])


def _tc_finalize(spart_f, bias2):
  half = ROWS2 * PAYW
  return pl.pallas_call(
      _k4_body,
      grid=(1,),
      in_specs=[pl.BlockSpec((half,), lambda i: (0,)),
                pl.BlockSpec((half,), lambda i: (1,)),
                pl.BlockSpec((1, F), lambda i: (0, 0))],
      out_specs=pl.BlockSpec((N_NODES // 4, 4 * F), lambda i: (0, 0)),
      out_shape=jax.ShapeDtypeStruct((N_NODES // 4, 4 * F), jnp.float32),
  )(spart_f, spart_f, bias2)


def kernel(feat, efeat, W_attn, b_attn, W_e1, b_e1, W_e2, b_e2, bias,
           edge_index, etype):
  src = edge_index[0].astype(jnp.int32)
  dst = edge_index[1].astype(jnp.int32)
  et = etype.astype(jnp.int32)
  hz_f = _sc_gather(feat, src, dst)
  wc0 = W_e1.reshape(F * F, F)
  wc1 = W_e2.reshape(F * F, F)
  bc0 = b_e1.reshape(F, F)
  bc1 = b_e2.reshape(F, F)
  wa = jnp.kron(jnp.eye(4, dtype=jnp.float32),
                jnp.tile(W_attn, (1, F)))
  ba = b_attn.reshape(1, 1)
  col = jnp.arange(F * F, dtype=jnp.int32)[None, :]
  row = jnp.arange(F, dtype=jnp.int32)[:, None]
  rm = (col // F == row).astype(jnp.float32)
  tm = (col % F == row).astype(jnp.float32)
  et8 = et.reshape(-1, 1)
  payload_f = _tc_dense(hz_f, efeat, et8, wa, ba,
                        wc0, wc1, bc0, bc1, rm, tm)
  spart_f = _sc_scatter(payload_f, dst, et)
  out64 = _tc_finalize(spart_f, bias.reshape(1, F))
  return out64.reshape(N_NODES, F)


# speedup vs baseline: 1.0076x; 1.0076x over previous
import jax
import jax.numpy as jnp
from jax import lax
from jax.experimental import pallas as pl
from jax.experimental.pallas import tpu as pltpu
from jax.experimental.pallas import tpu_sc as plsc

N_NODES = 10000
N_EDGES = 160000
F = 16
NEG_SLOPE = 0.01

NC, NS = 2, 16
NW = NC * NS
RQ = N_EDGES // 4
RB = 1600
NBLK = RQ // RB
BLK_E = 4 * RB
WR = 1248
WR_LAST = RQ - (NW - 1) * WR
RCH = 32
NCH = WR // RCH
NCH_LAST = WR_LAST // RCH
ROWS2 = 2 * N_NODES
SROWS = ROWS2 + 96
ZPT = SROWS // NS
WPT = 1248
PAYW = 2 * F

_mesh = plsc.VectorSubcoreMesh(
    core_axis_name="c", subcore_axis_name="s", num_cores=NC, num_subcores=NS)
_sc_params = pltpu.CompilerParams(use_tc_tiling_on_sc=False)


def _gather_body(feat_h, src_h, dst_h, hz_f,
                 idxs_v, idxd_v, gis0, gis1, gid0, gid1,
                 gh0, gh1, gz0, gz1, hzf0, hzf1,
                 sh0, sh1, sz0, sz1, so0, so1):
  c = lax.axis_index("c")
  s = lax.axis_index("s")
  w = c * NS + s
  bw = w * WR
  ifirst = bw // RB

  for ib in range(2):
    ibl = jnp.minimum(ifirst + ib, NBLK - 1)
    for j in range(4):
      pltpu.sync_copy(src_h.at[pl.ds(ibl * BLK_E + j * RB, RB)],
                      idxs_v.at[pl.ds((ib * 4 + j) * RB, RB)])
      pltpu.sync_copy(dst_h.at[pl.ds(ibl * BLK_E + j * RB, RB)],
                      idxd_v.at[pl.ds((ib * 4 + j) * RB, RB)])

  gis = (gis0, gis1)
  gid = (gid0, gid1)
  gh = (gh0, gh1)
  gz = (gz0, gz1)
  hzf = (hzf0, hzf1)
  hsem = (sh0, sh1)
  zsem = (sz0, sz1)
  osem = (so0, so1)

  def build_gidx(i, b):
    g0 = bw + i * RCH
    blk = g0 // RB
    base = (blk - ifirst) * 4 * RB + (g0 - blk * RB)
    for j in range(4):
      for h2 in range(2):
        off = base + j * RB + 16 * h2
        gis[b][pl.ds(32 * j + 16 * h2, 16)] = idxs_v[pl.ds(off, 16)]
        gid[b][pl.ds(32 * j + 16 * h2, 16)] = idxd_v[pl.ds(off, 16)]

  def start_gather(b):
    pltpu.async_copy(feat_h.at[gis[b]], gh[b], hsem[b])
    pltpu.async_copy(feat_h.at[gid[b]], gz[b], zsem[b])

  def wait_gather(b):
    pltpu.make_async_copy(feat_h.at[gis[b]], gh[b], hsem[b]).wait()
    pltpu.make_async_copy(feat_h.at[gid[b]], gz[b], zsem[b]).wait()

  def pack(b):
    for i2 in range(RCH):
      for j in range(4):
        hzf[b][pl.ds(128 * i2 + 32 * j, 16)] = gh[b][32 * j + i2, pl.ds(0, 16)]
        hzf[b][pl.ds(128 * i2 + 32 * j + 16, 16)] = gz[b][32 * j + i2, pl.ds(0, 16)]

  def start_out(i, b):
    pltpu.async_copy(
        hzf[b], hz_f.at[pl.ds((bw + i * RCH) * 128, RCH * 128)], osem[b])

  def wait_out(b):
    pltpu.make_async_copy(
        hzf[b], hz_f.at[pl.ds(bw * 128, RCH * 128)], osem[b]).wait()

  build_gidx(0, 0)
  start_gather(0)
  build_gidx(1, 1)
  start_gather(1)

  def pair(k, _):
    for b in range(2):
      i = 2 * k + b
      wait_gather(b)

      @pl.when(k > 0)
      def _wo():
        wait_out(b)

      pack(b)
      start_out(i, b)
      nxt = i + 2
      if b == 0:
        build_gidx(nxt, b)
        start_gather(b)
      else:
        @pl.when(k < (NCH - 1) // 2 - 1)
        def _ng():
          build_gidx(nxt, b)
          start_gather(b)
    return 0

  lax.fori_loop(0, (NCH - 1) // 2, pair, 0)

  wait_gather(0)
  wait_out(0)
  pack(0)
  start_out(NCH - 1, 0)
  wait_out(1)
  wait_out(0)

  @pl.when(w == NW - 1)
  def _extra():
    for i in (NCH, NCH + 1):
      build_gidx(i, 0)
      start_gather(0)
      wait_gather(0)
      pack(0)
      pltpu.sync_copy(hzf[0], hz_f.at[pl.ds((bw + i * RCH) * 128, RCH * 128)])


def _sc_gather(feat, src, dst):
  return pl.kernel(
      _gather_body,
      out_type=jax.ShapeDtypeStruct((N_EDGES * 32,), jnp.float32),
      mesh=_mesh,
      compiler_params=_sc_params,
      scratch_types=[
          pltpu.VMEM((8 * RB,), jnp.int32),
          pltpu.VMEM((8 * RB,), jnp.int32),
          pltpu.VMEM((128,), jnp.int32),
          pltpu.VMEM((128,), jnp.int32),
          pltpu.VMEM((128,), jnp.int32),
          pltpu.VMEM((128,), jnp.int32),
          pltpu.VMEM((128, F), jnp.float32),
          pltpu.VMEM((128, F), jnp.float32),
          pltpu.VMEM((128, F), jnp.float32),
          pltpu.VMEM((128, F), jnp.float32),
          pltpu.VMEM((RCH * 128,), jnp.float32),
          pltpu.VMEM((RCH * 128,), jnp.float32),
      ] + [pltpu.SemaphoreType.DMA] * 6,
  )(feat, src, dst)


def _k2_body(hz_ref, ef_ref, et_ref,
             wa_ref, ba_ref, wc0_ref, wc1_ref, bc0_ref, bc1_ref,
             rm_ref, tm_ref, out_ref):
  x = hz_ref[...].reshape(RB, 128)
  a = jnp.dot(x, wa_ref[...], preferred_element_type=jnp.float32) + ba_ref[...]
  a = jnp.where(a >= 0.0, a, NEG_SLOPE * a)
  exa = jnp.exp(a)
  efa = ef_ref[...]
  eta = et_ref[...]
  lane = lax.broadcasted_iota(jnp.int32, (RB, F), 1)
  parts = []
  for j in range(4):
    h = x[:, 32 * j:32 * j + F]
    ef = efa[j * RB:(j + 1) * RB]
    p = (jnp.dot(ef, rm_ref[...], preferred_element_type=jnp.float32) *
         jnp.dot(h, tm_ref[...], preferred_element_type=jnp.float32))
    v0 = (jnp.dot(p, wc0_ref[...], preferred_element_type=jnp.float32) +
          jnp.dot(h, bc0_ref[...], preferred_element_type=jnp.float32))
    v1 = (jnp.dot(p, wc1_ref[...], preferred_element_type=jnp.float32) +
          jnp.dot(h, bc1_ref[...], preferred_element_type=jnp.float32))
    m0 = (eta[j * RB:(j + 1) * RB] == 0).astype(jnp.float32)
    ex = exa[:, j:j + 1]
    msg = ex * (m0 * v0 + (1.0 - m0) * v1)
    exl = jnp.where(lane == 0, jnp.broadcast_to(ex, (RB, F)), 0.0)
    parts.append(msg)
    parts.append(exl)
  out = jnp.concatenate(parts, axis=1)
  out_ref[...] = out.reshape(RB * 128)


def _tc_dense(hz_f, efeat, et8, wa, ba, wc0, wc1, bc0, bc1, rm, tm):
  return pl.pallas_call(
      _k2_body,
      grid=(NBLK,),
      in_specs=[
          pl.BlockSpec((RB * 128,), lambda i: (i,)),
          pl.BlockSpec((BLK_E, F), lambda i: (i, 0)),
          pl.BlockSpec((BLK_E, 1), lambda i: (i, 0)),
          pl.BlockSpec((128, 4), lambda i: (0, 0)),
          pl.BlockSpec((1, 1), lambda i: (0, 0)),
          pl.BlockSpec((F * F, F), lambda i: (0, 0)),
          pl.BlockSpec((F * F, F), lambda i: (0, 0)),
          pl.BlockSpec((F, F), lambda i: (0, 0)),
          pl.BlockSpec((F, F), lambda i: (0, 0)),
          pl.BlockSpec((F, F * F), lambda i: (0, 0)),
          pl.BlockSpec((F, F * F), lambda i: (0, 0)),
      ],
      out_specs=pl.BlockSpec((RB * 128,), lambda i: (i,)),
      out_shape=jax.ShapeDtypeStruct((N_EDGES * 32,), jnp.float32),
  )(hz_f, efeat, et8, wa, ba, wc0, wc1, bc0, bc1, rm, tm)


def _scatter_body(pay_f, dst_h, et_h, spart_f,
                  idxd_v, idxt_v, r0, r1, pf0, pf1, sb0, sb1,
                  wb2d, wbf, sacc, ps0, ps1, cs0, cs1):
  c = lax.axis_index("c")
  s = lax.axis_index("s")
  w = c * NS + s
  bw = w * WR
  ifirst = bw // RB

  for i in range(128):
    sb0[i, pl.ds(0, 16)] = jnp.zeros((16,), jnp.float32)
    sb0[i, pl.ds(16, 16)] = jnp.zeros((16,), jnp.float32)
  for m in range(9):
    pltpu.sync_copy(sb0.at[pl.ds(0, 128)],
                    sacc.at[pl.ds(s * ZPT + m * 128, 128)])
  pltpu.sync_copy(sb0.at[pl.ds(0, ZPT - 9 * 128)],
                  sacc.at[pl.ds(s * ZPT + 9 * 128, ZPT - 9 * 128)])

  plsc.subcore_barrier()

  for ib in range(2):
    ibl = jnp.minimum(ifirst + ib, NBLK - 1)
    for j in range(4):
      pltpu.sync_copy(dst_h.at[pl.ds(ibl * BLK_E + j * RB, RB)],
                      idxd_v.at[pl.ds((ib * 4 + j) * RB, RB)])
      pltpu.sync_copy(et_h.at[pl.ds(ibl * BLK_E + j * RB, RB)],
                      idxt_v.at[pl.ds((ib * 4 + j) * RB, RB)])

  rb = (r0, r1)
  pf = (pf0, pf1)
  sb = (sb0, sb1)
  psem = (ps0, ps1)
  csem = (cs0, cs1)

  def build_ridx(i, b):
    g0 = bw + i * RCH
    blk = g0 // RB
    base = (blk - ifirst) * 4 * RB + (g0 - blk * RB)
    for j in range(4):
      for h2 in range(2):
        off = base + j * RB + 16 * h2
        d16 = idxd_v[pl.ds(off, 16)]
        t16 = idxt_v[pl.ds(off, 16)]
        rb[b][pl.ds(32 * j + 16 * h2, 16)] = d16 + t16 * N_NODES

  def start_pay(i, b):
    pltpu.async_copy(
        pay_f.at[pl.ds((bw + i * RCH) * 128, RCH * 128)], pf[b], psem[b])

  def wait_pay(b):
    pltpu.make_async_copy(
        pay_f.at[pl.ds(bw * 128, RCH * 128)], pf[b], psem[b]).wait()

  def unpack(b):
    for i2 in range(RCH):
      for j in range(4):
        sb[b][32 * j + i2, pl.ds(0, 16)] = pf[b][pl.ds(128 * i2 + 32 * j, 16)]
        sb[b][32 * j + i2, pl.ds(16, 16)] = pf[b][pl.ds(128 * i2 + 32 * j + 16, 16)]

  def start_scat(b):
    pltpu.async_copy(sb[b], sacc.at[rb[b]], csem[b], add=True)

  def wait_scat(b):
    pltpu.make_async_copy(sb[b], sacc.at[rb[b]], csem[b]).wait()

  start_pay(0, 0)
  start_pay(1, 1)

  def pair(k, _):
    for b in range(2):
      i = 2 * k + b
      wait_pay(b)

      @pl.when(k > 0)
      def _ws():
        wait_scat(b)

      unpack(b)
      build_ridx(i, b)
      start_scat(b)
      nxt = i + 2
      if b == 0:
        start_pay(nxt, b)
      else:
        @pl.when(k < (NCH - 1) // 2 - 1)
        def _np():
          start_pay(nxt, b)
    return 0

  lax.fori_loop(0, (NCH - 1) // 2, pair, 0)

  wait_pay(0)
  wait_scat(0)
  unpack(0)
  build_ridx(NCH - 1, 0)
  start_scat(0)
  wait_scat(1)
  wait_scat(0)

  @pl.when(w == NW - 1)
  def _extra():
    for i in (NCH, NCH + 1):
      start_pay(i, 0)
      wait_pay(0)
      unpack(0)
      build_ridx(i, 0)
      pltpu.sync_copy(sb[0], sacc.at[rb[0]], add=True)

  plsc.subcore_barrier()

  def wb_chunk(q0, nrows):
    pltpu.sync_copy(sacc.at[pl.ds(q0, nrows)], wb2d.at[pl.ds(0, nrows)])
    for r in range(nrows):
      wbf[pl.ds(32 * r, 16)] = wb2d[r, pl.ds(0, 16)]
      wbf[pl.ds(32 * r + 16, 16)] = wb2d[r, pl.ds(16, 16)]
    pltpu.sync_copy(wbf.at[pl.ds(0, nrows * 32)],
                    spart_f.at[pl.ds((c * ROWS2 + q0) * 32, nrows * 32)])

  @pl.when(s < NS - 1)
  def _wb():
    def wbody(m, _):
      wb_chunk(s * WPT + m * 96, 96)
      return 0
    lax.fori_loop(0, 13, wbody, 0)

  @pl.when(s == NS - 1)
  def _wb_last():
    def wbody(m, _):
      wb_chunk((NS - 1) * WPT + m * 128, 128)
      return 0
    lax.fori_loop(0, 10, wbody, 0)


def _sc_scatter(payload_f, dst, et):
  return pl.kernel(
      _scatter_body,
      out_type=jax.ShapeDtypeStruct((NC * ROWS2 * PAYW,), jnp.float32),
      mesh=_mesh,
      compiler_params=_sc_params,
      scratch_types=[
          pltpu.VMEM((8 * RB,), jnp.int32),
          pltpu.VMEM((8 * RB,), jnp.int32),
          pltpu.VMEM((128,), jnp.int32),
          pltpu.VMEM((128,), jnp.int32),
          pltpu.VMEM((RCH * 128,), jnp.float32),
          pltpu.VMEM((RCH * 128,), jnp.float32),
          pltpu.VMEM((128, PAYW), jnp.float32),
          pltpu.VMEM((128, PAYW), jnp.float32),
          pltpu.VMEM((128, PAYW), jnp.float32),
          pltpu.VMEM((128 * PAYW,), jnp.float32),
          pltpu.VMEM_SHARED((SROWS, PAYW), jnp.float32),
      ] + [pltpu.SemaphoreType.DMA] * 4,
  )(payload_f, dst, et)


def _k4_body(sp0_ref, sp1_ref, bias_ref, out_ref):
  x = (sp0_ref[...].reshape(ROWS2 // 4, 128) +
       sp1_ref[...].reshape(ROWS2 // 4, 128))
  x0 = x[0:N_NODES // 4]
  x1 = x[N_NODES // 4:2 * (N_NODES // 4)]
  for u in range(4):
    m0 = x0[:, 32 * u:32 * u + F]
    d0 = x0[:, 32 * u + F:32 * u + F + 1]
    m1 = x1[:, 32 * u:32 * u + F]
    d1 = x1[:, 32 * u + F:32 * u + F + 1]
    out_ref[:, F * u:F * (u + 1)] = (m0 / jnp.where(d0 > 0, d0, 1.0) +
                                     m1 / jnp.where(d1 > 0, d1, 1.0) +
                                     bias_ref[...])


def _tc_finalize(spart_f, bias2):
  half = ROWS2 * PAYW
  return pl.pallas_call(
      _k4_body,
      grid=(1,),
      in_specs=[pl.BlockSpec((half,), lambda i: (0,)),
                pl.BlockSpec((half,), lambda i: (1,)),
                pl.BlockSpec((1, F), lambda i: (0, 0))],
      out_specs=pl.BlockSpec((N_NODES // 4, 4 * F), lambda i: (0, 0)),
      out_shape=jax.ShapeDtypeStruct((N_NODES // 4, 4 * F), jnp.float32),
  )(spart_f, spart_f, bias2)


def kernel(feat, efeat, W_attn, b_attn, W_e1, b_e1, W_e2, b_e2, bias,
           edge_index, etype):
  src = edge_index[0].astype(jnp.int32)
  dst = edge_index[1].astype(jnp.int32)
  et = etype.astype(jnp.int32)
  hz_f = _sc_gather(feat, src, dst)
  wc0 = W_e1.reshape(F * F, F)
  wc1 = W_e2.reshape(F * F, F)
  bc0 = b_e1.reshape(F, F)
  bc1 = b_e2.reshape(F, F)
  wa = jnp.kron(jnp.eye(4, dtype=jnp.float32), W_attn)
  ba = b_attn.reshape(1, 1)
  col = jnp.arange(F * F, dtype=jnp.int32)[None, :]
  row = jnp.arange(F, dtype=jnp.int32)[:, None]
  rm = (col // F == row).astype(jnp.float32)
  tm = (col % F == row).astype(jnp.float32)
  et8 = et.reshape(-1, 1)
  payload_f = _tc_dense(hz_f, efeat, et8, wa, ba,
                        wc0, wc1, bc0, bc1, rm, tm)
  spart_f = _sc_scatter(payload_f, dst, et)
  out64 = _tc_finalize(spart_f, bias.reshape(1, F))
  return out64.reshape(N_NODES, F)
